# baseline (speedup 1.0000x reference)
import functools

import jax
import jax.numpy as jnp
from jax import lax
from jax.experimental import pallas as pl
from jax.experimental.pallas import tpu as pltpu
from jax.experimental.pallas import tpu_sc as plsc

NC = 2
NS = 16
NW = NC * NS
LANES = 16

CHUNK = 400
SUB = 80
NSUB = CHUNK // SUB


def _sc_embed(idx, emb_weight, pos_table, seq):
    T = idx.shape[0]
    D = emb_weight.shape[1]
    per_tile = T // NW
    n_chunks = per_tile // CHUNK
    assert T == per_tile * NW and per_tile == n_chunks * CHUNK
    assert CHUNK == 2 * seq and seq % LANES == 0 or CHUNK % seq == 0

    mesh = plsc.VectorSubcoreMesh(core_axis_name="c", subcore_axis_name="s")

    @functools.partial(
        pl.kernel,
        out_type=jax.ShapeDtypeStruct((T, D), jnp.float32),
        mesh=mesh,
        compiler_params=pltpu.CompilerParams(needs_layout_passes=False),
        scratch_types=[
            pltpu.VMEM((CHUNK,), jnp.int32),
            pltpu.VMEM((CHUNK,), jnp.int32),
            pltpu.VMEM((CHUNK, D), jnp.float32),
            pltpu.VMEM(pos_table.shape, jnp.float32),
            pltpu.VMEM((SUB, D), jnp.float32),
            pltpu.SemaphoreType.DMA,
            pltpu.SemaphoreType.DMA,
            pltpu.SemaphoreType.DMA,
        ],
    )
    def body(idx_hbm, tab_hbm, ptab_hbm, out_hbm, idx_v, pos_v, tok_r,
             posblk, pos_r, sem_t, sem_p, sem_o):
        wid = lax.axis_index("s") * NC + lax.axis_index("c")
        base = wid * per_tile
        pltpu.sync_copy(ptab_hbm, posblk)

        def chunk_body(c, carry):
            tbase = base + c * CHUNK
            pltpu.sync_copy(idx_hbm.at[pl.ds(tbase, CHUNK)], idx_v)

            def scan_body(j, z):
                t = idx_v[pl.ds(j * LANES, LANES)]
                return z | (t == 0)

            zmask = lax.fori_loop(
                0, CHUNK // LANES, scan_body,
                jnp.zeros((LANES,), jnp.bool_))
            zcnt = plsc.all_reduce_population_count(zmask)[0]

            @pl.when(c > 0)
            def _():
                pltpu.make_async_copy(
                    out_hbm.at[pl.ds(0, CHUNK)], tok_r, sem_o).wait()

            cps = [pltpu.async_copy(
                tab_hbm.at[idx_v.at[pl.ds(k * SUB, SUB)]],
                tok_r.at[pl.ds(k * SUB, SUB)], sem_t) for k in range(NSUB)]
            for cp in cps:
                cp.wait()

            @pl.when(zcnt == 0)
            def _():
                def add_body(i, carry2):
                    prow = jnp.where(i < seq, i + 1, i - (seq - 1))
                    for jj in range(D // LANES):
                        cols = pl.ds(jj * LANES, LANES)
                        plsc.addupdate(tok_r.at[i, cols], posblk[prow, cols])
                    return carry2

                lax.fori_loop(0, CHUNK, add_body, 0)

            @pl.when(zcnt > 0)
            def _():
                def pos_body(j, carry2):
                    t = idx_v[pl.ds(j * LANES, LANES)]
                    v = lax.iota(jnp.int32, LANES) + j * LANES
                    v = jnp.where(v >= seq, v - seq, v) + 1
                    pos_v[pl.ds(j * LANES, LANES)] = jnp.where(t == 0, 0, v)
                    return carry2

                lax.fori_loop(0, CHUNK // LANES, pos_body, 0)
                for k in range(NSUB):
                    pltpu.async_copy(
                        ptab_hbm.at[pos_v.at[pl.ds(k * SUB, SUB)]],
                        pos_r, sem_p).wait()

                    def add_body(i, carry2):
                        for jj in range(D // LANES):
                            cols = pl.ds(jj * LANES, LANES)
                            plsc.addupdate(tok_r.at[k * SUB + i, cols],
                                           pos_r[i, cols])
                        return carry2

                    lax.fori_loop(0, SUB, add_body, 0)

            pltpu.async_copy(tok_r, out_hbm.at[pl.ds(tbase, CHUNK)], sem_o)
            return carry

        lax.fori_loop(0, n_chunks, chunk_body, 0)
        pltpu.make_async_copy(
            out_hbm.at[pl.ds(0, CHUNK)], tok_r, sem_o).wait()

    return body(idx, emb_weight, pos_table)


def kernel(inputs, emb_weight, pos_table):
    B, L = inputs.shape
    D = emb_weight.shape[1]
    P = pos_table.shape[0]
    pad = (-P) % 8
    if pad:
        pos_table = jnp.concatenate(
            [pos_table, jnp.zeros((pad, D), pos_table.dtype)])
    out = _sc_embed(inputs.reshape(B * L), emb_weight, pos_table, L)
    return out.reshape(B, L, D)


# speedup vs baseline: 1.7746x; 1.7746x over previous
import functools

import jax
import jax.numpy as jnp
from jax import lax
from jax.experimental import pallas as pl
from jax.experimental.pallas import tpu as pltpu
from jax.experimental.pallas import tpu_sc as plsc

NC = 2
NS = 16
NW = NC * NS
LANES = 16

CHUNK = 400
SUB = 80
NSUB = CHUNK // SUB


def _sc_embed(idx, emb_weight, pos_table, seq):
    T = idx.shape[0]
    D = emb_weight.shape[1]
    per_tile = T // NW
    n_chunks = per_tile // CHUNK
    assert T == per_tile * NW and per_tile == n_chunks * CHUNK
    assert CHUNK % seq == 0

    mesh = plsc.VectorSubcoreMesh(core_axis_name="c", subcore_axis_name="s")

    @functools.partial(
        pl.kernel,
        out_type=jax.ShapeDtypeStruct((T, D), jnp.float32),
        mesh=mesh,
        compiler_params=pltpu.CompilerParams(needs_layout_passes=False),
        scratch_types=[
            pltpu.VMEM((CHUNK,), jnp.int32),
            pltpu.VMEM((CHUNK,), jnp.int32),
            pltpu.VMEM((CHUNK, D), jnp.float32),
            pltpu.VMEM((CHUNK, D), jnp.float32),
            pltpu.VMEM((SUB, D), jnp.float32),
            pltpu.SemaphoreType.DMA,
            pltpu.SemaphoreType.DMA,
            pltpu.SemaphoreType.DMA,
        ],
    )
    def body(idx_hbm, tab_hbm, ptab_hbm, out_hbm, idx_v, pos_v, tok_r,
             posblk, pos_r, sem_t, sem_p, sem_o):
        wid = lax.axis_index("s") * NC + lax.axis_index("c")
        base = wid * per_tile

        def pat_body(j, carry2):
            v = lax.iota(jnp.int32, LANES) + j * LANES
            pos_v[pl.ds(j * LANES, LANES)] = jnp.where(v >= seq, v - seq, v) + 1
            return carry2

        lax.fori_loop(0, CHUNK // LANES, pat_body, 0)
        for k in range(NSUB):
            pltpu.async_copy(
                ptab_hbm.at[pos_v.at[pl.ds(k * SUB, SUB)]],
                posblk.at[pl.ds(k * SUB, SUB)], sem_p).wait()

        def chunk_body(c, carry):
            tbase = base + c * CHUNK
            pltpu.sync_copy(idx_hbm.at[pl.ds(tbase, CHUNK)], idx_v)

            def scan_body(j, z):
                t = idx_v[pl.ds(j * LANES, LANES)]
                return z | (t == 0)

            zmask = lax.fori_loop(
                0, CHUNK // LANES, scan_body,
                jnp.zeros((LANES,), jnp.bool_))
            zcnt = plsc.all_reduce_population_count(zmask)[0]

            @pl.when(c > 0)
            def _():
                pltpu.make_async_copy(
                    out_hbm.at[pl.ds(0, CHUNK)], tok_r, sem_o).wait()

            cps = [pltpu.async_copy(
                tab_hbm.at[idx_v.at[pl.ds(k * SUB, SUB)]],
                tok_r.at[pl.ds(k * SUB, SUB)], sem_t) for k in range(NSUB)]
            for cp in cps:
                cp.wait()

            @pl.when(zcnt == 0)
            def _():
                def add_body(i, carry2):
                    for jj in range(D // LANES):
                        cols = pl.ds(jj * LANES, LANES)
                        plsc.addupdate(tok_r.at[i, cols], posblk[i, cols])
                    return carry2

                lax.fori_loop(0, CHUNK, add_body, 0)

            @pl.when(zcnt > 0)
            def _():
                def pos_body(j, carry2):
                    t = idx_v[pl.ds(j * LANES, LANES)]
                    v = lax.iota(jnp.int32, LANES) + j * LANES
                    v = jnp.where(v >= seq, v - seq, v) + 1
                    pos_v[pl.ds(j * LANES, LANES)] = jnp.where(t == 0, 0, v)
                    return carry2

                lax.fori_loop(0, CHUNK // LANES, pos_body, 0)
                for k in range(NSUB):
                    pltpu.async_copy(
                        ptab_hbm.at[pos_v.at[pl.ds(k * SUB, SUB)]],
                        pos_r, sem_p).wait()

                    def add_body(i, carry2):
                        for jj in range(D // LANES):
                            cols = pl.ds(jj * LANES, LANES)
                            plsc.addupdate(tok_r.at[k * SUB + i, cols],
                                           pos_r[i, cols])
                        return carry2

                    lax.fori_loop(0, SUB, add_body, 0)

                lax.fori_loop(0, CHUNK // LANES, pat_body, 0)

            pltpu.async_copy(tok_r, out_hbm.at[pl.ds(tbase, CHUNK)], sem_o)
            return carry

        lax.fori_loop(0, n_chunks, chunk_body, 0)
        pltpu.make_async_copy(
            out_hbm.at[pl.ds(0, CHUNK)], tok_r, sem_o).wait()

    return body(idx, emb_weight, pos_table)


def kernel(inputs, emb_weight, pos_table):
    B, L = inputs.shape
    D = emb_weight.shape[1]
    P = pos_table.shape[0]
    pad = (-P) % 8
    if pad:
        pos_table = jnp.concatenate(
            [pos_table, jnp.zeros((pad, D), pos_table.dtype)])
    out = _sc_embed(inputs.reshape(B * L), emb_weight, pos_table, L)
    return out.reshape(B, L, D)


# speedup vs baseline: 2.6416x; 1.4886x over previous
import functools

import jax
import jax.numpy as jnp
from jax import lax
from jax.experimental import pallas as pl
from jax.experimental.pallas import tpu as pltpu
from jax.experimental.pallas import tpu_sc as plsc

NC = 2
NS = 16
NW = NC * NS
LANES = 16

CHUNK = 200
SUBS = (104, 96)
SUB_OFF = (0, 104)


def _sc_embed(idx, emb_weight, pos_table, seq):
    T = idx.shape[0]
    D = emb_weight.shape[1]
    per_tile = T // NW
    n_chunks = per_tile // CHUNK
    assert T == per_tile * NW and per_tile == n_chunks * CHUNK
    assert CHUNK == seq and n_chunks % 2 == 0
    ngrp = (CHUNK + LANES - 1) // LANES

    mesh = plsc.VectorSubcoreMesh(core_axis_name="c", subcore_axis_name="s")

    @functools.partial(
        pl.kernel,
        out_type=jax.ShapeDtypeStruct((T, D), jnp.float32),
        mesh=mesh,
        compiler_params=pltpu.CompilerParams(needs_layout_passes=False),
        scratch_types=[
            pltpu.VMEM((per_tile,), jnp.int32),
            pltpu.VMEM((CHUNK,), jnp.int32),
            pltpu.VMEM((CHUNK, D), jnp.float32),
            pltpu.VMEM((CHUNK, D), jnp.float32),
            pltpu.VMEM((CHUNK, D), jnp.float32),
            pltpu.VMEM((SUBS[0], D), jnp.float32),
            pltpu.SemaphoreType.DMA,
            pltpu.SemaphoreType.DMA,
            pltpu.SemaphoreType.DMA,
            pltpu.SemaphoreType.DMA,
            pltpu.SemaphoreType.DMA,
        ],
    )
    def body(idx_hbm, tab_hbm, ptab_hbm, out_hbm, idx_v, pos_v, tok0, tok1,
             posblk, pos_r, sem_t0, sem_t1, sem_o0, sem_o1, sem_p):
        wid = lax.axis_index("s") * NC + lax.axis_index("c")
        base = wid * per_tile
        toks = (tok0, tok1)
        sem_t = (sem_t0, sem_t1)
        sem_o = (sem_o0, sem_o1)

        pltpu.sync_copy(idx_hbm.at[pl.ds(base, per_tile)], idx_v)

        def pat_body(j, carry2):
            off = jnp.minimum(j * LANES, CHUNK - LANES)
            v = lax.iota(jnp.int32, LANES) + off
            pos_v[pl.ds(off, LANES)] = jnp.where(v >= seq, v - seq, v) + 1
            return carry2

        lax.fori_loop(0, ngrp, pat_body, 0)
        for k, (ko, kl) in enumerate(zip(SUB_OFF, SUBS)):
            pltpu.async_copy(
                ptab_hbm.at[pos_v.at[pl.ds(ko, kl)]],
                posblk.at[pl.ds(ko, kl)], sem_p).wait()

        def scan_chunk(c):
            def scan_body(j, z):
                off = c * CHUNK + jnp.minimum(j * LANES, CHUNK - LANES)
                t = idx_v[pl.ds(off, LANES)]
                return z | (t == 0)

            zmask = lax.fori_loop(0, ngrp, scan_body,
                                  jnp.zeros((LANES,), jnp.bool_))
            return plsc.all_reduce_population_count(zmask)[0]

        def fire_gathers(c, b):
            for ko, kl in zip(SUB_OFF, SUBS):
                pltpu.async_copy(
                    tab_hbm.at[idx_v.at[pl.ds(c * CHUNK + ko, kl)]],
                    toks[b].at[pl.ds(ko, kl)], sem_t[b])

        def drain(sem, nbytes_ref):
            pltpu.make_async_copy(
                tab_hbm.at[pl.ds(0, CHUNK)], nbytes_ref, sem).wait()

        def prefetch(c_next, b_next):
            @pl.when(c_next < n_chunks)
            def _():
                @pl.when(c_next > 1)
                def _():
                    drain(sem_o[b_next], toks[b_next])
                fire_gathers(c_next, b_next)

        def add_chunk(c, b, zcnt):
            tok_r = toks[b]

            @pl.when(zcnt == 0)
            def _():
                def add_body(i, carry2):
                    for jj in range(D // LANES):
                        cols = pl.ds(jj * LANES, LANES)
                        plsc.addupdate(tok_r.at[i, cols], posblk[i, cols])
                    return carry2

                lax.fori_loop(0, CHUNK, add_body, 0)

            @pl.when(zcnt > 0)
            def _():
                def pos_body(j, carry2):
                    off = jnp.minimum(j * LANES, CHUNK - LANES)
                    t = idx_v[pl.ds(c * CHUNK + off, LANES)]
                    v = lax.iota(jnp.int32, LANES) + off
                    v = jnp.where(v >= seq, v - seq, v) + 1
                    pos_v[pl.ds(off, LANES)] = jnp.where(t == 0, 0, v)
                    return carry2

                lax.fori_loop(0, ngrp, pos_body, 0)
                for ko, kl in zip(SUB_OFF, SUBS):
                    pltpu.async_copy(
                        ptab_hbm.at[pos_v.at[pl.ds(ko, kl)]],
                        pos_r.at[pl.ds(0, kl)], sem_p).wait()

                    def add_body(i, carry2):
                        for jj in range(D // LANES):
                            cols = pl.ds(jj * LANES, LANES)
                            plsc.addupdate(tok_r.at[ko + i, cols],
                                           pos_r[i, cols])
                        return carry2

                    lax.fori_loop(0, kl, add_body, 0)

        def half(c, b, z_cur):
            z_next = scan_chunk(jnp.minimum(c + 1, n_chunks - 1))
            prefetch(c + 1, 1 - b)
            drain(sem_t[b], toks[b])
            add_chunk(c, b, z_cur)
            pltpu.async_copy(
                toks[b], out_hbm.at[pl.ds(base + c * CHUNK, CHUNK)], sem_o[b])
            return z_next

        z0 = scan_chunk(0)
        fire_gathers(0, 0)

        def super_body(k, z_even):
            z_odd = half(2 * k, 0, z_even)
            return half(2 * k + 1, 1, z_odd)

        lax.fori_loop(0, n_chunks // 2, super_body, z0)
        drain(sem_o0, tok0)
        drain(sem_o1, tok1)

    return body(idx, emb_weight, pos_table)


def kernel(inputs, emb_weight, pos_table):
    B, L = inputs.shape
    D = emb_weight.shape[1]
    P = pos_table.shape[0]
    pad = (-P) % 8
    if pad:
        pos_table = jnp.concatenate(
            [pos_table, jnp.zeros((pad, D), pos_table.dtype)])
    out = _sc_embed(inputs.reshape(B * L), emb_weight, pos_table, L)
    return out.reshape(B, L, D)


# speedup vs baseline: 3.0563x; 1.1570x over previous
import functools

import jax
import jax.numpy as jnp
from jax import lax
from jax.experimental import pallas as pl
from jax.experimental.pallas import tpu as pltpu
from jax.experimental.pallas import tpu_sc as plsc

NC = 2
NS = 16
NW = NC * NS
LANES = 16

CHUNK = 200
SUBS = (104, 96)
SUB_OFF = (0, 104)


def _sc_embed(idx, emb_weight, pos_table, seq):
    T = idx.shape[0]
    D = emb_weight.shape[1]
    per_tile = T // NW
    n_chunks = per_tile // CHUNK
    assert T == per_tile * NW and per_tile == n_chunks * CHUNK
    assert CHUNK == seq and n_chunks % 2 == 0
    ngrp = (CHUNK + LANES - 1) // LANES

    mesh = plsc.VectorSubcoreMesh(core_axis_name="c", subcore_axis_name="s")

    @functools.partial(
        pl.kernel,
        out_type=jax.ShapeDtypeStruct((T, D), jnp.float32),
        mesh=mesh,
        compiler_params=pltpu.CompilerParams(needs_layout_passes=False),
        scratch_types=[
            pltpu.VMEM((per_tile,), jnp.int32),
            pltpu.VMEM((CHUNK,), jnp.int32),
            pltpu.VMEM((CHUNK, D), jnp.float32),
            pltpu.VMEM((CHUNK, D), jnp.float32),
            pltpu.VMEM((CHUNK, D), jnp.float32),
            pltpu.VMEM((SUBS[0], D), jnp.float32),
            pltpu.SemaphoreType.DMA,
            pltpu.SemaphoreType.DMA,
            pltpu.SemaphoreType.DMA,
            pltpu.SemaphoreType.DMA,
            pltpu.SemaphoreType.DMA,
        ],
    )
    def body(idx_hbm, tab_hbm, ptab_hbm, out_hbm, idx_v, pos_v, tok0, tok1,
             posblk, pos_r, sem_t0, sem_t1, sem_o0, sem_o1, sem_p):
        wid = lax.axis_index("s") * NC + lax.axis_index("c")
        base = wid * per_tile
        toks = (tok0, tok1)
        sem_t = (sem_t0, sem_t1)
        sem_o = (sem_o0, sem_o1)

        pltpu.sync_copy(idx_hbm.at[pl.ds(base, per_tile)], idx_v)

        def pat_body(j, carry2):
            off = jnp.minimum(j * LANES, CHUNK - LANES)
            v = lax.iota(jnp.int32, LANES) + off
            pos_v[pl.ds(off, LANES)] = jnp.where(v >= seq, v - seq, v) + 1
            return carry2

        lax.fori_loop(0, ngrp, pat_body, 0)
        for k, (ko, kl) in enumerate(zip(SUB_OFF, SUBS)):
            pltpu.async_copy(
                ptab_hbm.at[pos_v.at[pl.ds(ko, kl)]],
                posblk.at[pl.ds(ko, kl)], sem_p).wait()

        def scan_chunk(c):
            def scan_body(j, z):
                off = c * CHUNK + jnp.minimum(j * LANES, CHUNK - LANES)
                t = idx_v[pl.ds(off, LANES)]
                return z | (t == 0)

            zmask = lax.fori_loop(0, ngrp, scan_body,
                                  jnp.zeros((LANES,), jnp.bool_))
            return plsc.all_reduce_population_count(zmask)[0]

        def fire_gathers(c, b):
            for ko, kl in zip(SUB_OFF, SUBS):
                pltpu.async_copy(
                    tab_hbm.at[idx_v.at[pl.ds(c * CHUNK + ko, kl)]],
                    toks[b].at[pl.ds(ko, kl)], sem_t[b])

        def drain(sem, nbytes_ref):
            pltpu.make_async_copy(
                tab_hbm.at[pl.ds(0, CHUNK)], nbytes_ref, sem).wait()

        def prefetch(c_next, b_next):
            @pl.when(c_next < n_chunks)
            def _():
                @pl.when(c_next > 1)
                def _():
                    drain(sem_o[b_next], toks[b_next])
                fire_gathers(c_next, b_next)

        def add_chunk(c, b, zcnt):
            tok_r = toks[b]

            @pl.when(zcnt == 0)
            def _():
                for ko, kl in zip(SUB_OFF, SUBS):
                    def add_body(i, carry2, ko=ko):
                        for jj in range(D // LANES):
                            cols = pl.ds(jj * LANES, LANES)
                            plsc.addupdate(tok_r.at[ko + i, cols],
                                           posblk[ko + i, cols])
                        return carry2

                    lax.fori_loop(0, kl, add_body, 0)
                    pltpu.async_copy(
                        tok_r.at[pl.ds(ko, kl)],
                        out_hbm.at[pl.ds(base + c * CHUNK + ko, kl)],
                        sem_o[b])

            @pl.when(zcnt > 0)
            def _():
                def pos_body(j, carry2):
                    off = jnp.minimum(j * LANES, CHUNK - LANES)
                    t = idx_v[pl.ds(c * CHUNK + off, LANES)]
                    v = lax.iota(jnp.int32, LANES) + off
                    v = jnp.where(v >= seq, v - seq, v) + 1
                    pos_v[pl.ds(off, LANES)] = jnp.where(t == 0, 0, v)
                    return carry2

                lax.fori_loop(0, ngrp, pos_body, 0)
                for ko, kl in zip(SUB_OFF, SUBS):
                    pltpu.async_copy(
                        ptab_hbm.at[pos_v.at[pl.ds(ko, kl)]],
                        pos_r.at[pl.ds(0, kl)], sem_p).wait()

                    def add_body(i, carry2):
                        for jj in range(D // LANES):
                            cols = pl.ds(jj * LANES, LANES)
                            plsc.addupdate(tok_r.at[ko + i, cols],
                                           pos_r[i, cols])
                        return carry2

                    lax.fori_loop(0, kl, add_body, 0)
                    pltpu.async_copy(
                        tok_r.at[pl.ds(ko, kl)],
                        out_hbm.at[pl.ds(base + c * CHUNK + ko, kl)],
                        sem_o[b])

        def half(c, b, z_cur):
            z_next = scan_chunk(jnp.minimum(c + 1, n_chunks - 1))
            prefetch(c + 1, 1 - b)
            drain(sem_t[b], toks[b])
            add_chunk(c, b, z_cur)
            return z_next

        z0 = scan_chunk(0)
        fire_gathers(0, 0)

        def super_body(k, z_even):
            z_odd = half(2 * k, 0, z_even)
            return half(2 * k + 1, 1, z_odd)

        lax.fori_loop(0, n_chunks // 2, super_body, z0)
        drain(sem_o0, tok0)
        drain(sem_o1, tok1)

    return body(idx, emb_weight, pos_table)


def kernel(inputs, emb_weight, pos_table):
    B, L = inputs.shape
    D = emb_weight.shape[1]
    P = pos_table.shape[0]
    pad = (-P) % 8
    if pad:
        pos_table = jnp.concatenate(
            [pos_table, jnp.zeros((pad, D), pos_table.dtype)])
    out = _sc_embed(inputs.reshape(B * L), emb_weight, pos_table, L)
    return out.reshape(B, L, D)


# speedup vs baseline: 3.1474x; 1.0298x over previous
import functools

import jax
import jax.numpy as jnp
from jax import lax
from jax.experimental import pallas as pl
from jax.experimental.pallas import tpu as pltpu
from jax.experimental.pallas import tpu_sc as plsc

NC = 2
NS = 16
NW = NC * NS
LANES = 16

CHUNK = 200
SUBS = (104, 96)
SUB_OFF = (0, 104)
OSUBS = (56, 48, 48, 48)
OSUB_OFF = (0, 56, 104, 152)


def _sc_embed(idx, emb_weight, pos_table, seq):
    T = idx.shape[0]
    D = emb_weight.shape[1]
    per_tile = T // NW
    n_chunks = per_tile // CHUNK
    assert T == per_tile * NW and per_tile == n_chunks * CHUNK
    assert CHUNK == seq and n_chunks % 2 == 0
    ngrp = (CHUNK + LANES - 1) // LANES

    mesh = plsc.VectorSubcoreMesh(core_axis_name="c", subcore_axis_name="s")

    @functools.partial(
        pl.kernel,
        out_type=jax.ShapeDtypeStruct((T, D), jnp.float32),
        mesh=mesh,
        compiler_params=pltpu.CompilerParams(needs_layout_passes=False),
        scratch_types=[
            pltpu.VMEM((per_tile,), jnp.int32),
            pltpu.VMEM((CHUNK,), jnp.int32),
            pltpu.VMEM((CHUNK, D), jnp.float32),
            pltpu.VMEM((CHUNK, D), jnp.float32),
            pltpu.VMEM((CHUNK, D), jnp.float32),
            pltpu.VMEM((SUBS[0], D), jnp.float32),
            pltpu.SemaphoreType.DMA,
            pltpu.SemaphoreType.DMA,
            pltpu.SemaphoreType.DMA,
            pltpu.SemaphoreType.DMA,
            pltpu.SemaphoreType.DMA,
        ],
    )
    def body(idx_hbm, tab_hbm, ptab_hbm, out_hbm, idx_v, pos_v, tok0, tok1,
             posblk, pos_r, sem_t0, sem_t1, sem_o0, sem_o1, sem_p):
        wid = lax.axis_index("s") * NC + lax.axis_index("c")
        base = wid * per_tile
        toks = (tok0, tok1)
        sem_t = (sem_t0, sem_t1)
        sem_o = (sem_o0, sem_o1)

        pltpu.sync_copy(idx_hbm.at[pl.ds(base, per_tile)], idx_v)

        def pat_body(j, carry2):
            off = jnp.minimum(j * LANES, CHUNK - LANES)
            v = lax.iota(jnp.int32, LANES) + off
            pos_v[pl.ds(off, LANES)] = jnp.where(v >= seq, v - seq, v) + 1
            return carry2

        lax.fori_loop(0, ngrp, pat_body, 0)
        for k, (ko, kl) in enumerate(zip(SUB_OFF, SUBS)):
            pltpu.async_copy(
                ptab_hbm.at[pos_v.at[pl.ds(ko, kl)]],
                posblk.at[pl.ds(ko, kl)], sem_p).wait()

        def scan_chunk(c):
            def scan_body(j, z):
                off = c * CHUNK + jnp.minimum(j * LANES, CHUNK - LANES)
                t = idx_v[pl.ds(off, LANES)]
                return z | (t == 0)

            zmask = lax.fori_loop(0, ngrp, scan_body,
                                  jnp.zeros((LANES,), jnp.bool_))
            return plsc.all_reduce_population_count(zmask)[0]

        def fire_gathers(c, b):
            for ko, kl in zip(SUB_OFF, SUBS):
                pltpu.async_copy(
                    tab_hbm.at[idx_v.at[pl.ds(c * CHUNK + ko, kl)]],
                    toks[b].at[pl.ds(ko, kl)], sem_t[b])

        def drain(sem, nbytes_ref):
            pltpu.make_async_copy(
                tab_hbm.at[pl.ds(0, CHUNK)], nbytes_ref, sem).wait()

        def prefetch(c_next, b_next):
            @pl.when(c_next < n_chunks)
            def _():
                @pl.when(c_next > 1)
                def _():
                    drain(sem_o[b_next], toks[b_next])
                fire_gathers(c_next, b_next)

        def add_chunk(c, b, zcnt):
            tok_r = toks[b]

            @pl.when(zcnt == 0)
            def _():
                for ko, kl in zip(OSUB_OFF, OSUBS):
                    def add_body(i, carry2, ko=ko):
                        for jj in range(D // LANES):
                            cols = pl.ds(jj * LANES, LANES)
                            plsc.addupdate(tok_r.at[ko + i, cols],
                                           posblk[ko + i, cols])
                        return carry2

                    lax.fori_loop(0, kl, add_body, 0)
                    pltpu.async_copy(
                        tok_r.at[pl.ds(ko, kl)],
                        out_hbm.at[pl.ds(base + c * CHUNK + ko, kl)],
                        sem_o[b])

            @pl.when(zcnt > 0)
            def _():
                def pos_body(j, carry2):
                    off = jnp.minimum(j * LANES, CHUNK - LANES)
                    t = idx_v[pl.ds(c * CHUNK + off, LANES)]
                    v = lax.iota(jnp.int32, LANES) + off
                    v = jnp.where(v >= seq, v - seq, v) + 1
                    pos_v[pl.ds(off, LANES)] = jnp.where(t == 0, 0, v)
                    return carry2

                lax.fori_loop(0, ngrp, pos_body, 0)
                for ko, kl in zip(SUB_OFF, SUBS):
                    pltpu.async_copy(
                        ptab_hbm.at[pos_v.at[pl.ds(ko, kl)]],
                        pos_r.at[pl.ds(0, kl)], sem_p).wait()

                    def add_body(i, carry2):
                        for jj in range(D // LANES):
                            cols = pl.ds(jj * LANES, LANES)
                            plsc.addupdate(tok_r.at[ko + i, cols],
                                           pos_r[i, cols])
                        return carry2

                    lax.fori_loop(0, kl, add_body, 0)
                    pltpu.async_copy(
                        tok_r.at[pl.ds(ko, kl)],
                        out_hbm.at[pl.ds(base + c * CHUNK + ko, kl)],
                        sem_o[b])

        def half(c, b, z_cur):
            z_next = scan_chunk(jnp.minimum(c + 1, n_chunks - 1))
            prefetch(c + 1, 1 - b)
            drain(sem_t[b], toks[b])
            add_chunk(c, b, z_cur)
            return z_next

        z0 = scan_chunk(0)
        fire_gathers(0, 0)

        def super_body(k, z_even):
            z_odd = half(2 * k, 0, z_even)
            return half(2 * k + 1, 1, z_odd)

        lax.fori_loop(0, n_chunks // 2, super_body, z0)
        drain(sem_o0, tok0)
        drain(sem_o1, tok1)

    return body(idx, emb_weight, pos_table)


def kernel(inputs, emb_weight, pos_table):
    B, L = inputs.shape
    D = emb_weight.shape[1]
    P = pos_table.shape[0]
    pad = (-P) % 8
    if pad:
        pos_table = jnp.concatenate(
            [pos_table, jnp.zeros((pad, D), pos_table.dtype)])
    out = _sc_embed(inputs.reshape(B * L), emb_weight, pos_table, L)
    return out.reshape(B, L, D)


# speedup vs baseline: 3.1598x; 1.0039x over previous
import functools

import jax
import jax.numpy as jnp
from jax import lax
from jax.experimental import pallas as pl
from jax.experimental.pallas import tpu as pltpu
from jax.experimental.pallas import tpu_sc as plsc

NC = 2
NS = 16
NW = NC * NS
LANES = 16

CHUNK = 200
SUBS = (200,)
SUB_OFF = (0,)
OSUBS = (56, 48, 48, 48)
OSUB_OFF = (0, 56, 104, 152)


def _sc_embed(idx, emb_weight, pos_table, seq):
    T = idx.shape[0]
    D = emb_weight.shape[1]
    per_tile = T // NW
    n_chunks = per_tile // CHUNK
    assert T == per_tile * NW and per_tile == n_chunks * CHUNK
    assert CHUNK == seq and n_chunks % 2 == 0
    ngrp = (CHUNK + LANES - 1) // LANES

    mesh = plsc.VectorSubcoreMesh(core_axis_name="c", subcore_axis_name="s")

    @functools.partial(
        pl.kernel,
        out_type=jax.ShapeDtypeStruct((T, D), jnp.float32),
        mesh=mesh,
        compiler_params=pltpu.CompilerParams(needs_layout_passes=False),
        scratch_types=[
            pltpu.VMEM((per_tile,), jnp.int32),
            pltpu.VMEM((CHUNK,), jnp.int32),
            pltpu.VMEM((CHUNK, D), jnp.float32),
            pltpu.VMEM((CHUNK, D), jnp.float32),
            pltpu.VMEM((CHUNK, D), jnp.float32),
            pltpu.VMEM((SUBS[0], D), jnp.float32),
            pltpu.SemaphoreType.DMA,
            pltpu.SemaphoreType.DMA,
            pltpu.SemaphoreType.DMA,
            pltpu.SemaphoreType.DMA,
            pltpu.SemaphoreType.DMA,
        ],
    )
    def body(idx_hbm, tab_hbm, ptab_hbm, out_hbm, idx_v, pos_v, tok0, tok1,
             posblk, pos_r, sem_t0, sem_t1, sem_o0, sem_o1, sem_p):
        wid = lax.axis_index("s") * NC + lax.axis_index("c")
        base = wid * per_tile
        toks = (tok0, tok1)
        sem_t = (sem_t0, sem_t1)
        sem_o = (sem_o0, sem_o1)

        pltpu.sync_copy(idx_hbm.at[pl.ds(base, per_tile)], idx_v)

        def pat_body(j, carry2):
            off = jnp.minimum(j * LANES, CHUNK - LANES)
            v = lax.iota(jnp.int32, LANES) + off
            pos_v[pl.ds(off, LANES)] = jnp.where(v >= seq, v - seq, v) + 1
            return carry2

        lax.fori_loop(0, ngrp, pat_body, 0)
        for k, (ko, kl) in enumerate(zip(SUB_OFF, SUBS)):
            pltpu.async_copy(
                ptab_hbm.at[pos_v.at[pl.ds(ko, kl)]],
                posblk.at[pl.ds(ko, kl)], sem_p).wait()

        def scan_chunk(c):
            def scan_body(j, z):
                off = c * CHUNK + jnp.minimum(j * LANES, CHUNK - LANES)
                t = idx_v[pl.ds(off, LANES)]
                return z | (t == 0)

            zmask = lax.fori_loop(0, ngrp, scan_body,
                                  jnp.zeros((LANES,), jnp.bool_))
            return plsc.all_reduce_population_count(zmask)[0]

        def fire_gathers(c, b):
            for ko, kl in zip(SUB_OFF, SUBS):
                pltpu.async_copy(
                    tab_hbm.at[idx_v.at[pl.ds(c * CHUNK + ko, kl)]],
                    toks[b].at[pl.ds(ko, kl)], sem_t[b])

        def drain(sem, nbytes_ref):
            pltpu.make_async_copy(
                tab_hbm.at[pl.ds(0, CHUNK)], nbytes_ref, sem).wait()

        def prefetch(c_next, b_next):
            @pl.when(c_next < n_chunks)
            def _():
                @pl.when(c_next > 1)
                def _():
                    drain(sem_o[b_next], toks[b_next])
                fire_gathers(c_next, b_next)

        def add_chunk(c, b, zcnt):
            tok_r = toks[b]

            @pl.when(zcnt == 0)
            def _():
                for ko, kl in zip(OSUB_OFF, OSUBS):
                    def add_body(i, carry2, ko=ko):
                        for jj in range(D // LANES):
                            cols = pl.ds(jj * LANES, LANES)
                            plsc.addupdate(tok_r.at[ko + i, cols],
                                           posblk[ko + i, cols])
                        return carry2

                    lax.fori_loop(0, kl, add_body, 0)
                    pltpu.async_copy(
                        tok_r.at[pl.ds(ko, kl)],
                        out_hbm.at[pl.ds(base + c * CHUNK + ko, kl)],
                        sem_o[b])

            @pl.when(zcnt > 0)
            def _():
                def pos_body(j, carry2):
                    off = jnp.minimum(j * LANES, CHUNK - LANES)
                    t = idx_v[pl.ds(c * CHUNK + off, LANES)]
                    v = lax.iota(jnp.int32, LANES) + off
                    v = jnp.where(v >= seq, v - seq, v) + 1
                    pos_v[pl.ds(off, LANES)] = jnp.where(t == 0, 0, v)
                    return carry2

                lax.fori_loop(0, ngrp, pos_body, 0)
                for ko, kl in zip(SUB_OFF, SUBS):
                    pltpu.async_copy(
                        ptab_hbm.at[pos_v.at[pl.ds(ko, kl)]],
                        pos_r.at[pl.ds(0, kl)], sem_p).wait()

                    def add_body(i, carry2):
                        for jj in range(D // LANES):
                            cols = pl.ds(jj * LANES, LANES)
                            plsc.addupdate(tok_r.at[ko + i, cols],
                                           pos_r[i, cols])
                        return carry2

                    lax.fori_loop(0, kl, add_body, 0)
                    pltpu.async_copy(
                        tok_r.at[pl.ds(ko, kl)],
                        out_hbm.at[pl.ds(base + c * CHUNK + ko, kl)],
                        sem_o[b])

        def half(c, b, z_cur):
            z_next = scan_chunk(jnp.minimum(c + 1, n_chunks - 1))
            prefetch(c + 1, 1 - b)
            drain(sem_t[b], toks[b])
            add_chunk(c, b, z_cur)
            return z_next

        z0 = scan_chunk(0)
        fire_gathers(0, 0)

        def super_body(k, z_even):
            z_odd = half(2 * k, 0, z_even)
            return half(2 * k + 1, 1, z_odd)

        lax.fori_loop(0, n_chunks // 2, super_body, z0)
        drain(sem_o0, tok0)
        drain(sem_o1, tok1)

    return body(idx, emb_weight, pos_table)


def kernel(inputs, emb_weight, pos_table):
    B, L = inputs.shape
    D = emb_weight.shape[1]
    P = pos_table.shape[0]
    pad = (-P) % 8
    if pad:
        pos_table = jnp.concatenate(
            [pos_table, jnp.zeros((pad, D), pos_table.dtype)])
    out = _sc_embed(inputs.reshape(B * L), emb_weight, pos_table, L)
    return out.reshape(B, L, D)
